# baseline (speedup 1.0000x reference)
import functools

import jax
import jax.numpy as jnp
from jax import lax
from jax.experimental import pallas as pl
from jax.experimental.pallas import tpu as pltpu
from jax.experimental.pallas import tpu_sc as plsc

_D = 64
_B = 16384
_NC = 2
_NS = 16
_NW = _NC * _NS
_BPW = _B // _NW
_PPW = _BPW // 2
_CH = 128
_NCH = _BPW // _CH


def _mse_body(rep_hbm, idx2_hbm, par_hbm, table_hbm, out_hbm,
              idx_v, par_v, rows_v, rep_v, acc_v, sem_g, sem_r):
    c = lax.axis_index("c")
    s = lax.axis_index("s")
    wid = s * _NC + c
    base = wid * _BPW

    pltpu.sync_copy(idx2_hbm.at[pl.ds(base, _BPW)], idx_v)
    pltpu.sync_copy(par_hbm.at[pl.ds(base, _BPW)], par_v)
    rep_cp = pltpu.async_copy(rep_hbm.at[pl.ds(wid * _PPW, _PPW)], rep_v,
                              sem_r)
    gathers = []
    for j in range(_NCH):
        gathers.append(pltpu.async_copy(
            table_hbm.at[idx_v.at[pl.ds(j * _CH, _CH)]],
            rows_v.at[pl.ds(j * _CH, _CH)], sem_g))
    rep_cp.wait()
    for g in gathers:
        g.wait()

    nk = _D // 16

    def one_row(b, rep_row, rep_off, accs):
        pbc = plsc.load_gather(par_v, [jnp.full((16,), b, jnp.int32)])
        hi = pbc != 0
        new = []
        for k in range(nk):
            e_lo = rows_v[b, pl.ds(k * 16, 16)]
            e_hi = rows_v[b, pl.ds(_D + k * 16, 16)]
            e = jnp.where(hi, e_hi, e_lo)
            r = rep_v[rep_row, pl.ds(rep_off + k * 16, 16)]
            d = e - r
            new.append(accs[k] + d * d)
        return tuple(new)

    def body(j, accs):
        accs = one_row(2 * j, j, 0, accs)
        accs = one_row(2 * j + 1, j, _D, accs)
        return accs

    zero = jnp.zeros((16,), jnp.float32)
    accs = lax.fori_loop(0, _PPW, body, (zero,) * nk)
    total = accs[0]
    for a in accs[1:]:
        total = total + a
    acc_v[...] = total * (1.0 / (_B * _D))
    pltpu.sync_copy(acc_v, out_hbm.at[wid])


@functools.partial(
    pl.kernel,
    out_type=jax.ShapeDtypeStruct((_NW, 16), jnp.float32),
    mesh=plsc.VectorSubcoreMesh(core_axis_name="c", subcore_axis_name="s"),
    compiler_params=pltpu.CompilerParams(use_tc_tiling_on_sc=True,
                                         needs_layout_passes=False),
    scratch_types=[
        pltpu.VMEM((_BPW,), jnp.int32),
        pltpu.VMEM((_BPW,), jnp.int32),
        pltpu.VMEM((_BPW, 2 * _D), jnp.float32),
        pltpu.VMEM((_PPW, 2 * _D), jnp.float32),
        pltpu.VMEM((16,), jnp.float32),
        pltpu.SemaphoreType.DMA,
        pltpu.SemaphoreType.DMA,
    ],
)
def _mse_sc(rep_hbm, idx2_hbm, par_hbm, table_hbm, out_hbm,
            idx_v, par_v, rows_v, rep_v, acc_v, sem_g, sem_r):
    _mse_body(rep_hbm, idx2_hbm, par_hbm, table_hbm, out_hbm,
              idx_v, par_v, rows_v, rep_v, acc_v, sem_g, sem_r)


def kernel(rep, expr, emb_weight):
    idx = expr.astype(jnp.int32)
    partials = _mse_sc(rep.reshape(_B // 2, 2 * _D), idx >> 1, idx & 1,
                       emb_weight.reshape(-1, 2 * _D))
    return jnp.sum(partials)


# speedup vs baseline: 1.0340x; 1.0340x over previous
import functools

import jax
import jax.numpy as jnp
from jax import lax
from jax.experimental import pallas as pl
from jax.experimental.pallas import tpu as pltpu
from jax.experimental.pallas import tpu_sc as plsc

_D = 64
_B = 16384
_NC = 2
_NS = 16
_NW = _NC * _NS
_BPW = _B // _NW
_CH = 128
_NCH = _BPW // _CH


def _mse_body(rep_hbm, idx_hbm, table_hbm, out_hbm,
              idx_v, rows_v, rep_v, acc_v, sem_g, sem_r):
    c = lax.axis_index("c")
    s = lax.axis_index("s")
    wid = s * _NC + c
    base = wid * _BPW

    pltpu.sync_copy(idx_hbm.at[pl.ds(base, _BPW)], idx_v)
    rep_cp = pltpu.async_copy(rep_hbm.at[pl.ds(base, _BPW)], rep_v, sem_r)
    gathers = []
    for j in range(_NCH):
        gathers.append(pltpu.async_copy(
            table_hbm.at[idx_v.at[pl.ds(j * _CH, _CH)]],
            rows_v.at[pl.ds(j * _CH, _CH)], sem_g))
    rep_cp.wait()
    for g in gathers:
        g.wait()

    def body(i, accs):
        new = []
        for k in range(_D // 16):
            r = rows_v[i, pl.ds(k * 16, 16)]
            t = rep_v[i, pl.ds(k * 16, 16)]
            d = r - t
            new.append(accs[k] + d * d)
        return tuple(new)

    zero = jnp.zeros((16,), jnp.float32)
    accs = lax.fori_loop(0, _BPW, body, (zero,) * (_D // 16))
    total = accs[0]
    for a in accs[1:]:
        total = total + a
    acc_v[...] = total * (1.0 / (_B * _D))
    pltpu.sync_copy(acc_v, out_hbm.at[wid])


@functools.partial(
    pl.kernel,
    out_type=jax.ShapeDtypeStruct((_NW, 16), jnp.float32),
    mesh=plsc.VectorSubcoreMesh(core_axis_name="c", subcore_axis_name="s"),
    compiler_params=pltpu.CompilerParams(
        use_tc_tiling_on_sc=False,
        skip_device_barrier=True,
        disable_semaphore_checks=True,
        disable_bounds_checks=True,
    ),
    scratch_types=[
        pltpu.VMEM((_BPW,), jnp.int32),
        pltpu.VMEM((_BPW, _D), jnp.float32),
        pltpu.VMEM((_BPW, _D), jnp.float32),
        pltpu.VMEM((16,), jnp.float32),
        pltpu.SemaphoreType.DMA,
        pltpu.SemaphoreType.DMA,
    ],
)
def _mse_sc(rep_hbm, idx_hbm, table_hbm, out_hbm,
            idx_v, rows_v, rep_v, acc_v, sem_g, sem_r):
    _mse_body(rep_hbm, idx_hbm, table_hbm, out_hbm,
              idx_v, rows_v, rep_v, acc_v, sem_g, sem_r)


def kernel(rep, expr, emb_weight):
    partials = _mse_sc(rep, expr.astype(jnp.int32), emb_weight)
    return jnp.sum(partials)


# speedup vs baseline: 1.1024x; 1.0661x over previous
import functools

import jax
import jax.numpy as jnp
from jax import lax
from jax.experimental import pallas as pl
from jax.experimental.pallas import tpu as pltpu
from jax.experimental.pallas import tpu_sc as plsc

_D = 64
_B = 16384
_NC = 2
_NS = 16
_NW = _NC * _NS
_BPW = _B // _NW
_PPW = _BPW // 2
_CH = 128
_NCH = _BPW // _CH


def _mse_body(rep_hbm, idx_hbm, table_hbm, out_hbm,
              idx_v, rows_v, rep_v, acc_v, sem_g, sem_r):
    c = lax.axis_index("c")
    s = lax.axis_index("s")
    wid = s * _NC + c
    base = wid * _BPW

    pltpu.sync_copy(idx_hbm.at[pl.ds(base, _BPW)], idx_v)
    rep_cp = pltpu.async_copy(rep_hbm.at[pl.ds(wid * _PPW, _PPW)], rep_v,
                              sem_r)
    gathers = []
    for j in range(_NCH):
        gathers.append(pltpu.async_copy(
            table_hbm.at[idx_v.at[pl.ds(j * _CH, _CH)]],
            rows_v.at[pl.ds(j * _CH, _CH)], sem_g))
    rep_cp.wait()
    for g in gathers:
        g.wait()

    nk = _D // 16

    def body(j, accs):
        new = list(accs)
        for k in range(nk):
            e0 = rows_v[2 * j, pl.ds(k * 16, 16)]
            r0 = rep_v[j, pl.ds(k * 16, 16)]
            d0 = e0 - r0
            e1 = rows_v[2 * j + 1, pl.ds(k * 16, 16)]
            r1 = rep_v[j, pl.ds(_D + k * 16, 16)]
            d1 = e1 - r1
            new[k] = new[k] + d0 * d0 + d1 * d1
        return tuple(new)

    zero = jnp.zeros((16,), jnp.float32)
    accs = lax.fori_loop(0, _PPW, body, (zero,) * nk)
    total = accs[0]
    for a in accs[1:]:
        total = total + a
    acc_v[...] = total * (1.0 / (_B * _D))
    pltpu.sync_copy(acc_v, out_hbm.at[wid])


@functools.partial(
    pl.kernel,
    out_type=jax.ShapeDtypeStruct((_NW, 16), jnp.float32),
    mesh=plsc.VectorSubcoreMesh(core_axis_name="c", subcore_axis_name="s"),
    compiler_params=pltpu.CompilerParams(use_tc_tiling_on_sc=True),
    scratch_types=[
        pltpu.VMEM((_BPW,), jnp.int32),
        pltpu.VMEM((_BPW, 2 * _D), jnp.float32),
        pltpu.VMEM((_PPW, 2 * _D), jnp.float32),
        pltpu.VMEM((16,), jnp.float32),
        pltpu.SemaphoreType.DMA,
        pltpu.SemaphoreType.DMA,
    ],
)
def _mse_sc(rep_hbm, idx_hbm, table_hbm, out_hbm,
            idx_v, rows_v, rep_v, acc_v, sem_g, sem_r):
    _mse_body(rep_hbm, idx_hbm, table_hbm, out_hbm,
              idx_v, rows_v, rep_v, acc_v, sem_g, sem_r)


def kernel(rep, expr, emb_weight):
    table128 = jnp.pad(emb_weight, ((0, 0), (0, _D)))
    partials = _mse_sc(rep.reshape(_B // 2, 2 * _D), expr.astype(jnp.int32),
                       table128)
    return jnp.sum(partials)
